# baseline (speedup 1.0000x reference)
import functools

import jax
import jax.numpy as jnp
from jax import lax
from jax.experimental import pallas as pl
from jax.experimental.pallas import tpu as pltpu
from jax.experimental.pallas import tpu_sc as plsc

VOCAB = 100000
EMBED = 64
CTX = 20
BATCH = 1024
HIDDEN = 512

NC = 2
NS = 16
NW = NC * NS
NIDX = BATCH * CTX
BPW = NIDX // NW
CHUNK = 128
NCHUNK = BPW // CHUNK

VT = 2048
NVT = (VOCAB + VT - 1) // VT
NEG = -1e30


def _sc_gather(table, idx):
    @functools.partial(
        pl.kernel,
        mesh=plsc.VectorSubcoreMesh(core_axis_name="c", subcore_axis_name="s"),
        out_type=jax.ShapeDtypeStruct((NIDX, 2 * EMBED), jnp.float32),
        scratch_types=[
            pltpu.VMEM((BPW,), jnp.int32),
            pltpu.VMEM((BPW, 2 * EMBED), jnp.float32),
            pltpu.SemaphoreType.DMA,
        ],
    )
    def gather_kernel(table_hbm, idx_hbm, out_hbm, idx_v, rows_v, sem):
        wid = lax.axis_index("s") * NC + lax.axis_index("c")
        base = wid * BPW
        pltpu.sync_copy(idx_hbm.at[pl.ds(base, BPW)], idx_v)
        copies = [
            pltpu.async_copy(
                table_hbm.at[idx_v.at[pl.ds(c * CHUNK, CHUNK)]],
                rows_v.at[pl.ds(c * CHUNK, CHUNK)],
                sem,
            )
            for c in range(NCHUNK)
        ]
        for cp in copies:
            cp.wait()
        pltpu.sync_copy(rows_v, out_hbm.at[pl.ds(base, BPW)])

    return gather_kernel(table, idx)


def _select_half(g, par):
    def body(g_ref, p_ref, o_ref):
        o_ref[...] = jnp.where(
            p_ref[...] == 0, g_ref[:, :EMBED], g_ref[:, EMBED:]
        )

    return pl.pallas_call(
        body,
        out_shape=jax.ShapeDtypeStruct((NIDX, EMBED), jnp.float32),
    )(g, par)


def _mlp1(x, W1, b1r):
    def body(x_ref, w1_ref, b1_ref, h_ref):
        acc = jnp.dot(
            x_ref[...].astype(jnp.bfloat16),
            w1_ref[...].astype(jnp.bfloat16),
            preferred_element_type=jnp.float32,
        )
        h_ref[...] = jnp.maximum(acc + b1_ref[...], 0.0)

    return pl.pallas_call(
        body,
        out_shape=jax.ShapeDtypeStruct((BATCH, HIDDEN), jnp.float32),
    )(x, W1, b1r)


def _stats(h, W2, b2r):
    def body(h_ref, w2_ref, b2_ref, m_ref, s_ref):
        j = pl.program_id(0)
        logits = (
            jnp.dot(
                h_ref[...].astype(jnp.bfloat16),
                w2_ref[...].astype(jnp.bfloat16),
                preferred_element_type=jnp.float32,
            )
            + b2_ref[...]
        )
        col = j * VT + lax.broadcasted_iota(jnp.int32, (1, VT), 1)
        logits = jnp.where(col < VOCAB, logits, NEG)
        tmax = jnp.max(logits, axis=1, keepdims=True)
        first = j == 0
        m_old = jnp.where(first, NEG, m_ref[...])
        s_old = jnp.where(first, 0.0, s_ref[...])
        m_new = jnp.maximum(m_old, tmax)
        s_new = s_old * jnp.exp(m_old - m_new) + jnp.sum(
            jnp.exp(logits - m_new), axis=1, keepdims=True
        )
        m_ref[...] = m_new
        s_ref[...] = s_new

    return pl.pallas_call(
        body,
        grid=(NVT,),
        in_specs=[
            pl.BlockSpec((BATCH, HIDDEN), lambda j: (0, 0)),
            pl.BlockSpec((HIDDEN, VT), lambda j: (0, j)),
            pl.BlockSpec((1, VT), lambda j: (0, j)),
        ],
        out_specs=[
            pl.BlockSpec((BATCH, 1), lambda j: (0, 0)),
            pl.BlockSpec((BATCH, 1), lambda j: (0, 0)),
        ],
        out_shape=[
            jax.ShapeDtypeStruct((BATCH, 1), jnp.float32),
            jax.ShapeDtypeStruct((BATCH, 1), jnp.float32),
        ],
    )(h, W2, b2r)


def _final(h, W2, b2r, m, s):
    def body(h_ref, w2_ref, b2_ref, m_ref, s_ref, out_ref):
        logits = (
            jnp.dot(
                h_ref[...].astype(jnp.bfloat16),
                w2_ref[...].astype(jnp.bfloat16),
                preferred_element_type=jnp.float32,
            )
            + b2_ref[...]
        )
        out_ref[...] = logits - (m_ref[...] + jnp.log(s_ref[...]))

    return pl.pallas_call(
        body,
        grid=(NVT,),
        in_specs=[
            pl.BlockSpec((BATCH, HIDDEN), lambda j: (0, 0)),
            pl.BlockSpec((HIDDEN, VT), lambda j: (0, j)),
            pl.BlockSpec((1, VT), lambda j: (0, j)),
            pl.BlockSpec((BATCH, 1), lambda j: (0, 0)),
            pl.BlockSpec((BATCH, 1), lambda j: (0, 0)),
        ],
        out_specs=pl.BlockSpec((BATCH, VT), lambda j: (0, j)),
        out_shape=jax.ShapeDtypeStruct((BATCH, VOCAB), jnp.float32),
    )(h, W2, b2r, m, s)


def kernel(inputs, emb, W1, b1, W2, b2):
    flat_idx = inputs.reshape(-1).astype(jnp.int32)
    pairs = _sc_gather(emb.reshape(VOCAB // 2, 2 * EMBED), flat_idx >> 1)
    embeds = _select_half(pairs, (flat_idx & 1).reshape(NIDX, 1))
    x = embeds.reshape(BATCH, CTX * EMBED)
    h = _mlp1(x, W1, b1.reshape(1, HIDDEN))
    b2r = b2.reshape(1, VOCAB)
    m, s = _stats(h, W2, b2r)
    return _final(h, W2, b2r, m, s)
